# baseline (speedup 1.0000x reference)
import jax
import jax.numpy as jnp
from jax import lax
from jax.experimental import pallas as pl
from jax.experimental.pallas import tpu as pltpu
from jax.experimental.pallas import tpu_sc as plsc

N = 10000
F_IN = 256
H1 = 32
E = 160000
NUM_PROTS = 100

NC = 2
NS = 16
NW = NC * NS
NPAD = 10240
SLICE = NPAD // NS
EPW = 5120
EPAD = NW * EPW
CH = 128
K = EPW // CH
NBL = 5
GRP = K // NBL
RB = 1000


def _sc_deg_body(dst_hbm, zeros_hbm, out_hbm, dstb_v, acc_v, tmp_v, part_s):
    c = lax.axis_index("c")
    s = lax.axis_index("s")
    wid = s * NC + c
    pltpu.sync_copy(dst_hbm.at[wid], dstb_v)
    pltpu.sync_copy(zeros_hbm, acc_v)
    ones = jnp.ones((16,), jnp.float32)

    def step(i, carry):
        d = dstb_v[pl.ds(i * 16, 16)]
        plsc.addupdate_scatter(acc_v, [d], ones)
        return carry

    lax.fori_loop(0, EPW // 16, step, 0)
    pltpu.sync_copy(acc_v, part_s.at[s])
    plsc.subcore_barrier()
    for k2 in range(NS):
        pltpu.sync_copy(part_s.at[k2, pl.ds(s * SLICE, SLICE)],
                        tmp_v.at[pl.ds(k2 * SLICE, SLICE)])

    def csum(j, carry):
        o = j * 16
        v = tmp_v[pl.ds(o, 16)]
        for k2 in range(1, NS):
            v = v + tmp_v[pl.ds(k2 * SLICE + o, 16)]
        acc_v[pl.ds(o, 16)] = v
        return carry

    lax.fori_loop(0, SLICE // 16, csum, 0)
    pltpu.sync_copy(acc_v.at[pl.ds(0, SLICE)], out_hbm.at[c, pl.ds(s * SLICE, SLICE)])


def _sc_l2_body(src_hbm, dst_hbm, p_hbm, dinv_hbm, b1_hbm, w3_hbm, zeros_hbm,
                out_hbm, srcb_v, dstb_v, pa_v, pb_v, dinv_v, b1_v, w3_v, g2sl_v,
                g2_v, acc_v, tmp_v, g2_s, part_s):
    c = lax.axis_index("c")
    s = lax.axis_index("s")
    wid = s * NC + c
    base = s * SLICE
    pltpu.sync_copy(src_hbm.at[wid], srcb_v)
    pltpu.sync_copy(dst_hbm.at[wid], dstb_v)
    pltpu.sync_copy(p_hbm.at[0, pl.ds(base, SLICE)], pa_v)
    pltpu.sync_copy(p_hbm.at[1, pl.ds(base, SLICE)], pb_v)
    pltpu.sync_copy(dinv_hbm.at[pl.ds(base, SLICE)], dinv_v)
    pltpu.sync_copy(b1_hbm, b1_v)
    pltpu.sync_copy(w3_hbm, w3_v)
    lanes = lax.iota(jnp.int32, 16)

    def dmath(j, carry):
        r0 = j * 16
        idx_r = r0 + lanes
        dinv16 = dinv_v[pl.ds(r0, 16)]
        acc = jnp.zeros((16,), jnp.float32)
        for col in range(H1):
            idx_c = jnp.full((16,), col, jnp.int32)
            hv = (plsc.load_gather(pa_v, [idx_r, idx_c])
                  + plsc.load_gather(pb_v, [idx_r, idx_c]))
            hv = jnp.maximum(hv * dinv16 + b1_v[pl.ds(col * 16, 16)], 0.0)
            acc = acc + hv * w3_v[pl.ds(col * 16, 16)]
        g2sl_v[pl.ds(r0, 16)] = dinv16 * acc
        return carry

    lax.fori_loop(0, SLICE // 16, dmath, 0)
    pltpu.sync_copy(g2sl_v, g2_s.at[pl.ds(base, SLICE)])
    plsc.subcore_barrier()
    pltpu.sync_copy(g2_s, g2_v)
    pltpu.sync_copy(zeros_hbm, acc_v)

    def step(i, carry):
        sv = srcb_v[pl.ds(i * 16, 16)]
        d = dstb_v[pl.ds(i * 16, 16)]
        v = plsc.load_gather(g2_v, [sv])
        plsc.addupdate_scatter(acc_v, [d], v)
        return carry

    lax.fori_loop(0, EPW // 16, step, 0)
    pltpu.sync_copy(acc_v, part_s.at[s])
    plsc.subcore_barrier()
    for k2 in range(NS):
        pltpu.sync_copy(part_s.at[k2, pl.ds(s * SLICE, SLICE)],
                        tmp_v.at[pl.ds(k2 * SLICE, SLICE)])

    def csum(j, carry):
        o = j * 16
        v = tmp_v[pl.ds(o, 16)]
        for k2 in range(1, NS):
            v = v + tmp_v[pl.ds(k2 * SLICE + o, 16)]
        acc_v[pl.ds(o, 16)] = v
        return carry

    lax.fori_loop(0, SLICE // 16, csum, 0)
    pltpu.sync_copy(acc_v.at[pl.ds(0, SLICE)], out_hbm.at[c, pl.ds(s * SLICE, SLICE)])


def _sc_edge_body(tab_hbm, src_hbm, dst_hbm, zeros_hbm, out_hbm,
                  srcb, dstb, rows, stage, acc_s, tab_s, semA, semB):
    c = lax.axis_index("c")
    s = lax.axis_index("s")
    wid = s * NC + c
    pltpu.sync_copy(src_hbm.at[wid], srcb)
    pltpu.sync_copy(dst_hbm.at[wid], dstb)
    pltpu.sync_copy(tab_hbm.at[pl.ds(s * SLICE, SLICE)], stage)
    pltpu.sync_copy(stage, tab_s.at[pl.ds(s * SLICE, SLICE)])
    pltpu.sync_copy(zeros_hbm, stage)
    pltpu.sync_copy(stage, acc_s.at[pl.ds(s * SLICE, SLICE)])
    plsc.subcore_barrier()

    def issue(g, half, sem):
        base = g * NBL
        for b in range(NBL):
            pltpu.async_copy(tab_s.at[srcb.at[base + b]], rows.at[half, b], sem)

    def wait_n(half, sem):
        for b in range(NBL):
            pltpu.make_async_copy(tab_hbm.at[srcb.at[0]], rows.at[half, b], sem).wait()

    def scat(g, half):
        base = g * NBL
        for b in range(NBL):
            pltpu.sync_copy(rows.at[half, b], acc_s.at[dstb.at[base + b]], add=True)

    issue(0, 0, semA)

    def dbl(gg, carry):
        g0 = 2 * gg
        issue(g0 + 1, 1, semB)
        wait_n(0, semA)
        scat(g0, 0)

        @pl.when(g0 + 2 < GRP)
        def _():
            issue(g0 + 2, 0, semA)

        wait_n(1, semB)
        scat(g0 + 1, 1)
        return carry

    lax.fori_loop(0, GRP // 2, dbl, 0)
    plsc.subcore_barrier()
    pltpu.sync_copy(acc_s.at[pl.ds(s * SLICE, SLICE)], stage)
    pltpu.sync_copy(stage, out_hbm.at[c, pl.ds(s * SLICE, SLICE)])


def _tc_b0_body(x_ref, w_ref, h_ref):
    h_ref[...] = jnp.dot(x_ref[...], w_ref[...], preferred_element_type=jnp.float32)


def _tc_b1_body(h_ref, degp_ref, vmask_ref, g1_ref, dinv_ref):
    deg = degp_ref[0] + degp_ref[1]
    dinv = jnp.where(deg > 0.0, lax.rsqrt(deg), 0.0) * vmask_ref[...]
    g1_ref[...] = h_ref[...] * dinv[:, None]
    dinv_ref[...] = dinv


def _tc_f_body(q_ref, dinv_ref, b3_ref, out_ref):
    out_ref[...] = dinv_ref[...] * (q_ref[0, :] + q_ref[1, :]) + b3_ref[...]


def kernel(x, edge_index, batch, edge_attr, W1, b1, W3, b3):
    f32 = jnp.float32
    src = edge_index[0]
    dst = edge_index[1]
    pad_e = EPAD - E
    srcp = jnp.concatenate([src, jnp.full((pad_e,), N, jnp.int32)])
    dstp = jnp.concatenate([dst, jnp.full((pad_e,), N, jnp.int32)])
    src2 = srcp.reshape(NW, EPW)
    dst2 = dstp.reshape(NW, EPW)
    src3 = srcp.reshape(NW, K, CH)
    dst3 = dstp.reshape(NW, K, CH)
    vmask = jnp.zeros((NPAD,), f32).at[:N].set(1.0)
    zeros_n = jnp.zeros((NPAD,), f32)
    zeros_h1 = jnp.zeros((SLICE, H1), f32)

    mesh = plsc.VectorSubcoreMesh(core_axis_name="c", subcore_axis_name="s")
    sc_params = pltpu.CompilerParams(use_tc_tiling_on_sc=False, needs_layout_passes=False)

    degp = pl.kernel(
        _sc_deg_body,
        out_type=jax.ShapeDtypeStruct((NC, NPAD), f32),
        mesh=mesh,
        compiler_params=sc_params,
        scratch_types=[
            pltpu.VMEM((EPW,), jnp.int32),
            pltpu.VMEM((NPAD,), f32),
            pltpu.VMEM((NS * SLICE,), f32),
            pltpu.VMEM_SHARED((NS, NPAD), f32),
        ],
    )(dst2, zeros_n)

    h1 = pl.pallas_call(
        _tc_b0_body,
        grid=(N // RB,),
        in_specs=[
            pl.BlockSpec((RB, F_IN), lambda i: (i, 0)),
            pl.BlockSpec((F_IN, H1), lambda i: (0, 0)),
        ],
        out_specs=pl.BlockSpec((RB, H1), lambda i: (i, 0)),
        out_shape=jax.ShapeDtypeStruct((N, H1), f32),
    )(x, W1)

    h1p = jnp.pad(h1, ((0, NPAD - N), (0, 0)))
    g1, dinv = pl.pallas_call(
        _tc_b1_body,
        in_specs=[
            pl.BlockSpec((NPAD, H1), lambda: (0, 0)),
            pl.BlockSpec((NC, NPAD), lambda: (0, 0)),
            pl.BlockSpec((NPAD,), lambda: (0,)),
        ],
        out_specs=[
            pl.BlockSpec((NPAD, H1), lambda: (0, 0)),
            pl.BlockSpec((NPAD,), lambda: (0,)),
        ],
        out_shape=[
            jax.ShapeDtypeStruct((NPAD, H1), f32),
            jax.ShapeDtypeStruct((NPAD,), f32),
        ],
    )(h1p, degp, vmask)

    p = pl.kernel(
        _sc_edge_body,
        out_type=jax.ShapeDtypeStruct((NC, NPAD, H1), f32),
        mesh=mesh,
        compiler_params=sc_params,
        scratch_types=[
            pltpu.VMEM((K, CH), jnp.int32),
            pltpu.VMEM((K, CH), jnp.int32),
            pltpu.VMEM((2, NBL, CH, H1), f32),
            pltpu.VMEM((SLICE, H1), f32),
            pltpu.VMEM_SHARED((NPAD, H1), f32),
            pltpu.VMEM_SHARED((NPAD, H1), f32),
            pltpu.SemaphoreType.DMA,
            pltpu.SemaphoreType.DMA,
        ],
    )(g1, src3, dst3, zeros_h1)

    b1bc = jnp.broadcast_to(b1[:, None], (H1, 16)).reshape(-1)
    w3bc = jnp.broadcast_to(W3, (H1, 16)).reshape(-1)
    q = pl.kernel(
        _sc_l2_body,
        out_type=jax.ShapeDtypeStruct((NC, NPAD), f32),
        mesh=mesh,
        compiler_params=sc_params,
        scratch_types=[
            pltpu.VMEM((EPW,), jnp.int32),
            pltpu.VMEM((EPW,), jnp.int32),
            pltpu.VMEM((SLICE, H1), f32),
            pltpu.VMEM((SLICE, H1), f32),
            pltpu.VMEM((SLICE,), f32),
            pltpu.VMEM((H1 * 16,), f32),
            pltpu.VMEM((H1 * 16,), f32),
            pltpu.VMEM((SLICE,), f32),
            pltpu.VMEM((NPAD,), f32),
            pltpu.VMEM((NPAD,), f32),
            pltpu.VMEM((NS * SLICE,), f32),
            pltpu.VMEM_SHARED((NPAD,), f32),
            pltpu.VMEM_SHARED((NS, NPAD), f32),
        ],
    )(src2, dst2, p, dinv, b1bc, w3bc, zeros_n)

    out_full = pl.pallas_call(
        _tc_f_body,
        out_shape=jax.ShapeDtypeStruct((NPAD,), f32),
    )(q, dinv, jnp.broadcast_to(b3, (NPAD,)))

    return out_full[:N].reshape(-1, NUM_PROTS)


# speedup vs baseline: 1.0853x; 1.0853x over previous
import jax
import jax.numpy as jnp
from jax import lax
from jax.experimental import pallas as pl
from jax.experimental.pallas import tpu as pltpu
from jax.experimental.pallas import tpu_sc as plsc

N = 10000
F_IN = 256
H1 = 32
E = 160000
NUM_PROTS = 100

NC = 2
NS = 16
NW = NC * NS
NPAD = 10240
SLICE = NPAD // NS
EPW = 5120
EPAD = NW * EPW
CH = 128
K = EPW // CH
NBL = 5
GRP = K // NBL
RB = 1000


def _sc_deg_body(dst_hbm, zeros_hbm, out_hbm, dstb_v, acc_v, tmp_v, part_s):
    c = lax.axis_index("c")
    s = lax.axis_index("s")
    wid = s * NC + c
    pltpu.sync_copy(dst_hbm.at[wid], dstb_v)
    pltpu.sync_copy(zeros_hbm, acc_v)
    ones = jnp.ones((16,), jnp.float32)

    def step(i, carry):
        d = dstb_v[pl.ds(i * 16, 16)]
        plsc.addupdate_scatter(acc_v, [d], ones)
        return carry

    lax.fori_loop(0, EPW // 16, step, 0)
    pltpu.sync_copy(acc_v, part_s.at[s])
    plsc.subcore_barrier()
    for k2 in range(NS):
        pltpu.sync_copy(part_s.at[k2, pl.ds(s * SLICE, SLICE)],
                        tmp_v.at[pl.ds(k2 * SLICE, SLICE)])

    def csum(j, carry):
        o = j * 16
        v = tmp_v[pl.ds(o, 16)]
        for k2 in range(1, NS):
            v = v + tmp_v[pl.ds(k2 * SLICE + o, 16)]
        acc_v[pl.ds(o, 16)] = v
        return carry

    lax.fori_loop(0, SLICE // 16, csum, 0)
    pltpu.sync_copy(acc_v.at[pl.ds(0, SLICE)], out_hbm.at[c, pl.ds(s * SLICE, SLICE)])


def _sc_l2_body(src_hbm, dst_hbm, g2_hbm, zeros_hbm, out_hbm,
                srcb_v, dstb_v, g2_v, acc_v, tmp_v, part_s):
    c = lax.axis_index("c")
    s = lax.axis_index("s")
    wid = s * NC + c
    pltpu.sync_copy(src_hbm.at[wid], srcb_v)
    pltpu.sync_copy(dst_hbm.at[wid], dstb_v)
    pltpu.sync_copy(g2_hbm, g2_v)
    pltpu.sync_copy(zeros_hbm, acc_v)

    def step(i, carry):
        sv = srcb_v[pl.ds(i * 16, 16)]
        d = dstb_v[pl.ds(i * 16, 16)]
        v = plsc.load_gather(g2_v, [sv])
        plsc.addupdate_scatter(acc_v, [d], v)
        return carry

    lax.fori_loop(0, EPW // 16, step, 0)
    pltpu.sync_copy(acc_v, part_s.at[s])
    plsc.subcore_barrier()
    for k2 in range(NS):
        pltpu.sync_copy(part_s.at[k2, pl.ds(s * SLICE, SLICE)],
                        tmp_v.at[pl.ds(k2 * SLICE, SLICE)])

    def csum(j, carry):
        o = j * 16
        v = tmp_v[pl.ds(o, 16)]
        for k2 in range(1, NS):
            v = v + tmp_v[pl.ds(k2 * SLICE + o, 16)]
        acc_v[pl.ds(o, 16)] = v
        return carry

    lax.fori_loop(0, SLICE // 16, csum, 0)
    pltpu.sync_copy(acc_v.at[pl.ds(0, SLICE)], out_hbm.at[c, pl.ds(s * SLICE, SLICE)])


def _sc_edge_body(tab_hbm, src_hbm, dst_hbm, zeros_hbm, out_hbm,
                  srcb, dstb, rows, stage, acc_s, tab_s, semA, semB):
    c = lax.axis_index("c")
    s = lax.axis_index("s")
    wid = s * NC + c
    pltpu.sync_copy(src_hbm.at[wid], srcb)
    pltpu.sync_copy(dst_hbm.at[wid], dstb)
    pltpu.sync_copy(tab_hbm.at[pl.ds(s * SLICE, SLICE)], stage)
    pltpu.sync_copy(stage, tab_s.at[pl.ds(s * SLICE, SLICE)])
    pltpu.sync_copy(zeros_hbm, stage)
    pltpu.sync_copy(stage, acc_s.at[pl.ds(s * SLICE, SLICE)])
    plsc.subcore_barrier()

    def issue(g, half, sem):
        base = g * NBL
        for b in range(NBL):
            pltpu.async_copy(tab_s.at[srcb.at[base + b]], rows.at[half, b], sem)

    def wait_n(half, sem):
        for b in range(NBL):
            pltpu.make_async_copy(tab_hbm.at[srcb.at[0]], rows.at[half, b], sem).wait()

    def scat(g, half):
        base = g * NBL
        for b in range(NBL):
            pltpu.sync_copy(rows.at[half, b], acc_s.at[dstb.at[base + b]], add=True)

    issue(0, 0, semA)

    def dbl(gg, carry):
        g0 = 2 * gg
        issue(g0 + 1, 1, semB)
        wait_n(0, semA)
        scat(g0, 0)

        @pl.when(g0 + 2 < GRP)
        def _():
            issue(g0 + 2, 0, semA)

        wait_n(1, semB)
        scat(g0 + 1, 1)
        return carry

    lax.fori_loop(0, GRP // 2, dbl, 0)
    plsc.subcore_barrier()
    pltpu.sync_copy(acc_s.at[pl.ds(s * SLICE, SLICE)], stage)
    pltpu.sync_copy(stage, out_hbm.at[c, pl.ds(s * SLICE, SLICE)])


def _tc_b0_body(x_ref, w_ref, h_ref):
    h_ref[...] = jnp.dot(x_ref[...], w_ref[...], preferred_element_type=jnp.float32)


def _tc_b1_body(h_ref, degp_ref, vmask_ref, g1_ref, dinv_ref):
    deg = degp_ref[0] + degp_ref[1]
    dinv = jnp.where(deg > 0.0, lax.rsqrt(deg), 0.0) * vmask_ref[...]
    g1_ref[...] = h_ref[...] * dinv[:, None]
    dinv_ref[...] = dinv


def _tc_d_body(p_ref, dinv_ref, b1_ref, w3_ref, g2_ref):
    acc = p_ref[0] + p_ref[1]
    dinv = dinv_ref[...]
    h = jnp.maximum(acc * dinv[:, None] + b1_ref[...], 0.0)
    g2_ref[...] = dinv * jnp.sum(h * w3_ref[...], axis=1)


def _tc_f_body(q_ref, dinv_ref, b3_ref, out_ref):
    out_ref[...] = dinv_ref[...] * (q_ref[0, :] + q_ref[1, :]) + b3_ref[...]


def kernel(x, edge_index, batch, edge_attr, W1, b1, W3, b3):
    f32 = jnp.float32
    src = edge_index[0]
    dst = edge_index[1]
    pad_e = EPAD - E
    srcp = jnp.concatenate([src, jnp.full((pad_e,), N, jnp.int32)])
    dstp = jnp.concatenate([dst, jnp.full((pad_e,), N, jnp.int32)])
    src2 = srcp.reshape(NW, EPW)
    dst2 = dstp.reshape(NW, EPW)
    src3 = srcp.reshape(NW, K, CH)
    dst3 = dstp.reshape(NW, K, CH)
    vmask = jnp.zeros((NPAD,), f32).at[:N].set(1.0)
    zeros_n = jnp.zeros((NPAD,), f32)
    zeros_h1 = jnp.zeros((SLICE, H1), f32)

    mesh = plsc.VectorSubcoreMesh(core_axis_name="c", subcore_axis_name="s")
    sc_params = pltpu.CompilerParams(use_tc_tiling_on_sc=False, needs_layout_passes=False)

    degp = pl.kernel(
        _sc_deg_body,
        out_type=jax.ShapeDtypeStruct((NC, NPAD), f32),
        mesh=mesh,
        compiler_params=sc_params,
        scratch_types=[
            pltpu.VMEM((EPW,), jnp.int32),
            pltpu.VMEM((NPAD,), f32),
            pltpu.VMEM((NS * SLICE,), f32),
            pltpu.VMEM_SHARED((NS, NPAD), f32),
        ],
    )(dst2, zeros_n)

    h1 = pl.pallas_call(
        _tc_b0_body,
        grid=(N // RB,),
        in_specs=[
            pl.BlockSpec((RB, F_IN), lambda i: (i, 0)),
            pl.BlockSpec((F_IN, H1), lambda i: (0, 0)),
        ],
        out_specs=pl.BlockSpec((RB, H1), lambda i: (i, 0)),
        out_shape=jax.ShapeDtypeStruct((N, H1), f32),
    )(x, W1)

    h1p = jnp.pad(h1, ((0, NPAD - N), (0, 0)))
    g1, dinv = pl.pallas_call(
        _tc_b1_body,
        in_specs=[
            pl.BlockSpec((NPAD, H1), lambda: (0, 0)),
            pl.BlockSpec((NC, NPAD), lambda: (0, 0)),
            pl.BlockSpec((NPAD,), lambda: (0,)),
        ],
        out_specs=[
            pl.BlockSpec((NPAD, H1), lambda: (0, 0)),
            pl.BlockSpec((NPAD,), lambda: (0,)),
        ],
        out_shape=[
            jax.ShapeDtypeStruct((NPAD, H1), f32),
            jax.ShapeDtypeStruct((NPAD,), f32),
        ],
    )(h1p, degp, vmask)

    p = pl.kernel(
        _sc_edge_body,
        out_type=jax.ShapeDtypeStruct((NC, NPAD, H1), f32),
        mesh=mesh,
        compiler_params=sc_params,
        scratch_types=[
            pltpu.VMEM((K, CH), jnp.int32),
            pltpu.VMEM((K, CH), jnp.int32),
            pltpu.VMEM((2, NBL, CH, H1), f32),
            pltpu.VMEM((SLICE, H1), f32),
            pltpu.VMEM_SHARED((NPAD, H1), f32),
            pltpu.VMEM_SHARED((NPAD, H1), f32),
            pltpu.SemaphoreType.DMA,
            pltpu.SemaphoreType.DMA,
        ],
    )(g1, src3, dst3, zeros_h1)

    g2 = pl.pallas_call(
        _tc_d_body,
        out_shape=jax.ShapeDtypeStruct((NPAD,), f32),
    )(p, dinv, b1.reshape(1, H1), W3.reshape(1, H1))

    q = pl.kernel(
        _sc_l2_body,
        out_type=jax.ShapeDtypeStruct((NC, NPAD), f32),
        mesh=mesh,
        compiler_params=sc_params,
        scratch_types=[
            pltpu.VMEM((EPW,), jnp.int32),
            pltpu.VMEM((EPW,), jnp.int32),
            pltpu.VMEM((NPAD,), f32),
            pltpu.VMEM((NPAD,), f32),
            pltpu.VMEM((NS * SLICE,), f32),
            pltpu.VMEM_SHARED((NS, NPAD), f32),
        ],
    )(src2, dst2, g2, zeros_n)

    out_full = pl.pallas_call(
        _tc_f_body,
        out_shape=jax.ShapeDtypeStruct((NPAD,), f32),
    )(q, dinv, jnp.broadcast_to(b3, (NPAD,)))

    return out_full[:N].reshape(-1, NUM_PROTS)
